# Initial kernel scaffold; baseline (speedup 1.0000x reference)
import jax
import jax.numpy as jnp
from jax.experimental import pallas as pl


def kernel(y_pred, y_true):
    raise NotImplementedError("write your pallas kernel here")


# speedup vs baseline: 6.0498x; 6.0498x over previous
import math

import jax
import jax.numpy as jnp
from jax.experimental import pallas as pl
from jax.experimental.pallas import tpu as pltpu

_PAD = 0
_CONF = 0.9
_N = 2048
_V = 32000
_W = 640
_GRID = _V // _W

_L1 = _CONF
_L0 = (1.0 - _CONF) / (_V - 2)
_C = _L1 * math.log(_L1) + (_V - 1) * _L0 * math.log(_L0)


def _body(yt_ref, yp_ref, loss_ref, npad_ref):
    j = pl.program_id(0)
    yt = yt_ref[...]
    nonpad = yt != _PAD
    yp = yp_ref[...]
    col = jax.lax.broadcasted_iota(jnp.int32, (_N, _W), 1) + j * _W
    w = jnp.where(col == yt, _L1, _L0)
    w = jnp.where(nonpad, w, 0.0)
    part = jnp.sum(w * yp)

    @pl.when(j == 0)
    def _():
        npad_ref[0, 0] = jnp.sum(nonpad.astype(jnp.int32))
        loss_ref[0, 0] = 0.0

    loss_ref[0, 0] = loss_ref[0, 0] - part

    @pl.when(j == _GRID - 1)
    def _():
        loss_ref[0, 0] = (
            loss_ref[0, 0] + npad_ref[0, 0].astype(jnp.float32) * _C
        )


def kernel(y_pred, y_true):
    yp = y_pred.reshape(_N, _V)
    yt = y_true.reshape(_N, 1)
    loss, npad = pl.pallas_call(
        _body,
        grid=(_GRID,),
        in_specs=[
            pl.BlockSpec((_N, 1), lambda j: (0, 0)),
            pl.BlockSpec((_N, _W), lambda j: (0, j)),
        ],
        out_specs=[
            pl.BlockSpec(memory_space=pltpu.SMEM),
            pl.BlockSpec(memory_space=pltpu.SMEM),
        ],
        out_shape=[
            jax.ShapeDtypeStruct((1, 1), jnp.float32),
            jax.ShapeDtypeStruct((1, 1), jnp.int32),
        ],
    )(yt, yp)
    return (loss[0, 0], npad[0, 0])
